# baseline (speedup 1.0000x reference)
import functools

import jax
import jax.numpy as jnp
from jax import lax
from jax.experimental import pallas as pl
from jax.experimental.pallas import tpu as pltpu
from jax.experimental.pallas import tpu_sc as plsc

_NQ = 4
_NSUB = 16
_ZROWS = 64


def _stage1_body(x_ref, wg_ref, bg_ref, wf_ref, bf_ref, e_ref, f_ref):
    xb = x_ref[...]
    dn = (((1,), (1,)), ((), ()))
    lg = lax.dot_general(xb, wg_ref[...], dn, preferred_element_type=jnp.float32)
    e = jnp.exp(lg + bg_ref[...])
    fx = lax.dot_general(xb, wf_ref[...], dn, preferred_element_type=jnp.float32)
    fx = fx + bf_ref[...]
    ch = e.shape[1] // _NQ
    for k in range(_NQ):
        e_ref[k] = e[:, k * ch:(k + 1) * ch]
        f_ref[k] = fx[:, k * ch:(k + 1) * ch]


def _stage1(xf, Wg, bg, Wf, bf, blk):
    M, C = xf.shape
    ch = C // _NQ
    grid = (M // blk,)
    out_sds = jax.ShapeDtypeStruct((_NQ, M, ch), jnp.float32)
    return pl.pallas_call(
        _stage1_body,
        grid=grid,
        in_specs=[
            pl.BlockSpec((blk, C), lambda i: (i, 0)),
            pl.BlockSpec((C, C), lambda i: (0, 0)),
            pl.BlockSpec((1, C), lambda i: (0, 0)),
            pl.BlockSpec((C, C), lambda i: (0, 0)),
            pl.BlockSpec((1, C), lambda i: (0, 0)),
        ],
        out_specs=[
            pl.BlockSpec((_NQ, blk, ch), lambda i: (0, i, 0)),
            pl.BlockSpec((_NQ, blk, ch), lambda i: (0, i, 0)),
        ],
        out_shape=[out_sds, out_sds],
    )(xf, Wg, bg.reshape(1, C), Wf, bf.reshape(1, C))


def _sc_body(B, N, ch, rpt,
             e_hbm, f_hbm, ix_hbm, y0_hbm,
             ea_v, eb_v, fa_v, fb_v, d_v, z_v, idx_v, denom_sp, y0_sp,
             sem_ea, sem_eb, sem_fa, sem_fb, sem_d, sem_z):
    c = lax.axis_index("c")
    s = lax.axis_index("s")
    base = s * rpt
    nu = 2 * B
    e_bufs, f_bufs = (ea_v, eb_v), (fa_v, fb_v)
    sem_e, sem_f = (sem_ea, sem_eb), (sem_fa, sem_fb)

    def row0(u):
        b, qq = u // 2, u % 2
        return ((2 * c + qq) * B + b) * N + base

    pltpu.sync_copy(ix_hbm.at[pl.ds(base, rpt)], idx_v)

    zv16 = jnp.zeros((16,), jnp.float32)

    def _zero_row(r, _):
        for j in range(ch // 16):
            z_v[r, pl.ds(j * 16, 16)] = zv16
        return 0

    lax.fori_loop(0, _ZROWS, _zero_row, 0)

    he = pltpu.async_copy(e_hbm.at[pl.ds(row0(0), rpt), :], ea_v, sem_ea)
    hf = pltpu.async_copy(f_hbm.at[pl.ds(row0(0), rpt), :], fa_v, sem_fa)
    hzd = []
    for t in range(rpt // _ZROWS):
        sl = pl.ds(base + t * _ZROWS, _ZROWS)
        hzd.append(pltpu.async_copy(z_v, denom_sp.at[sl], sem_z))
    h_out = None

    for u in range(nu):
        e_v, f_v = e_bufs[u % 2], f_bufs[u % 2]

        def _vec_body(r, _):
            for j in range(ch // 16):
                sl = pl.ds(j * 16, 16)
                e = e_v[r, sl]
                d = d_v[r, sl]
                f = f_v[r, sl]
                e_v[r, sl] = f * (e / jnp.maximum(d, 1e-6))
            return 0

        if h_out is not None:
            h_out.wait()
        hzy = []
        for t in range(rpt // _ZROWS):
            sl = pl.ds(base + t * _ZROWS, _ZROWS)
            hzy.append(pltpu.async_copy(z_v, y0_sp.at[sl], sem_z))
        for h in hzd:
            h.wait()
        for h in hzy:
            h.wait()
        he.wait()

        plsc.subcore_barrier()

        pltpu.sync_copy(e_v, denom_sp.at[idx_v], add=True)

        plsc.subcore_barrier()

        hd = pltpu.async_copy(denom_sp.at[pl.ds(base, rpt)], d_v, sem_d)
        if u + 1 < nu:
            r1 = row0(u + 1)
            he = pltpu.async_copy(
                e_hbm.at[pl.ds(r1, rpt), :], e_bufs[(u + 1) % 2],
                sem_e[(u + 1) % 2])
            hf_n = pltpu.async_copy(
                f_hbm.at[pl.ds(r1, rpt), :], f_bufs[(u + 1) % 2],
                sem_f[(u + 1) % 2])
        hf.wait()
        hd.wait()

        hzd = []
        if u + 1 < nu:
            hf = hf_n
            for t in range(rpt // _ZROWS):
                sl = pl.ds(base + t * _ZROWS, _ZROWS)
                hzd.append(pltpu.async_copy(z_v, denom_sp.at[sl], sem_z))

        lax.fori_loop(0, rpt, _vec_body, 0)

        pltpu.sync_copy(e_v, y0_sp.at[idx_v], add=True)

        plsc.subcore_barrier()
        h_out = pltpu.async_copy(
            y0_sp.at[pl.ds(base, rpt)], y0_hbm.at[pl.ds(row0(u), rpt), :],
            sem_d)

    h_out.wait()


def _sc_stage(e2f, f2f, ix, B, N, ch):
    rpt = N // _NSUB
    mesh = plsc.VectorSubcoreMesh(
        core_axis_name="c", subcore_axis_name="s", num_cores=2,
        num_subcores=_NSUB)
    body = functools.partial(_sc_body, B, N, ch, rpt)
    sck = pl.kernel(
        body,
        mesh=mesh,
        out_type=jax.ShapeDtypeStruct((_NQ * B * N, ch), jnp.float32),
        scratch_types=[
            pltpu.VMEM((rpt, ch), jnp.float32),
            pltpu.VMEM((rpt, ch), jnp.float32),
            pltpu.VMEM((rpt, ch), jnp.float32),
            pltpu.VMEM((rpt, ch), jnp.float32),
            pltpu.VMEM((rpt, ch), jnp.float32),
            pltpu.VMEM((_ZROWS, ch), jnp.float32),
            pltpu.VMEM((rpt,), jnp.int32),
            pltpu.VMEM_SHARED((N, ch), jnp.float32),
            pltpu.VMEM_SHARED((N, ch), jnp.float32),
            pltpu.SemaphoreType.DMA,
            pltpu.SemaphoreType.DMA,
            pltpu.SemaphoreType.DMA,
            pltpu.SemaphoreType.DMA,
            pltpu.SemaphoreType.DMA,
            pltpu.SemaphoreType.DMA,
        ],
    )
    return sck(e2f, f2f, ix)


def _stage2_body(y0_ref, wh_ref, bh_ref, o_ref):
    yb = jnp.concatenate([y0_ref[k] for k in range(_NQ)], axis=1)
    dn = (((1,), (1,)), ((), ()))
    o = lax.dot_general(yb, wh_ref[...], dn, preferred_element_type=jnp.float32)
    o_ref[...] = o + bh_ref[...]


def _stage2(y04, Wh, bh, blk):
    _, M, ch = y04.shape
    C = _NQ * ch
    grid = (M // blk,)
    return pl.pallas_call(
        _stage2_body,
        grid=grid,
        in_specs=[
            pl.BlockSpec((_NQ, blk, ch), lambda i: (0, i, 0)),
            pl.BlockSpec((C, C), lambda i: (0, 0)),
            pl.BlockSpec((1, C), lambda i: (0, 0)),
        ],
        out_specs=pl.BlockSpec((blk, C), lambda i: (i, 0)),
        out_shape=jax.ShapeDtypeStruct((M, C), jnp.float32),
    )(y04, Wh, bh.reshape(1, C))


def kernel(x, ix, Wf, bf, Wg, bg, Wh, bh):
    B, N, C = x.shape
    ch = C // _NQ
    blk = 512
    hb = B // 2
    ys = []
    for g in range(2):
        xf = x[g * hb:(g + 1) * hb].reshape(hb * N, C)
        e4, f4 = _stage1(xf, Wg, bg, Wf, bf, blk)
        e4f = e4.reshape(_NQ * hb * N, ch)
        f4f = f4.reshape(_NQ * hb * N, ch)
        y0f = _sc_stage(e4f, f4f, ix, hb, N, ch)
        y04 = y0f.reshape(_NQ, hb * N, ch)
        ys.append(_stage2(y04, Wh, bh, blk))
    y = jnp.concatenate(ys, axis=0)
    return y.reshape(B, N, C)


# speedup vs baseline: 1.0881x; 1.0881x over previous
import functools

import jax
import jax.numpy as jnp
from jax import lax
from jax.experimental import pallas as pl
from jax.experimental.pallas import tpu as pltpu
from jax.experimental.pallas import tpu_sc as plsc

_NQ = 4
_NSUB = 16
_ZROWS = 64


def _stage1_body(x_ref, wg_ref, bg_ref, wf_ref, bf_ref, e_ref, f_ref):
    xb = x_ref[...]
    dn = (((1,), (1,)), ((), ()))
    lg = lax.dot_general(xb, wg_ref[...], dn, preferred_element_type=jnp.float32)
    e = jnp.exp(lg + bg_ref[...])
    fx = lax.dot_general(xb, wf_ref[...], dn, preferred_element_type=jnp.float32)
    fx = fx + bf_ref[...]
    ch = e.shape[1] // _NQ
    for k in range(_NQ):
        e_ref[k] = e[:, k * ch:(k + 1) * ch]
        f_ref[k] = fx[:, k * ch:(k + 1) * ch]


def _stage1(xf, Wg, bg, Wf, bf, blk):
    M, C = xf.shape
    ch = C // _NQ
    grid = (M // blk,)
    out_sds = jax.ShapeDtypeStruct((_NQ, M, ch), jnp.float32)
    return pl.pallas_call(
        _stage1_body,
        grid=grid,
        in_specs=[
            pl.BlockSpec((blk, C), lambda i: (i, 0)),
            pl.BlockSpec((C, C), lambda i: (0, 0)),
            pl.BlockSpec((1, C), lambda i: (0, 0)),
            pl.BlockSpec((C, C), lambda i: (0, 0)),
            pl.BlockSpec((1, C), lambda i: (0, 0)),
        ],
        out_specs=[
            pl.BlockSpec((_NQ, blk, ch), lambda i: (0, i, 0)),
            pl.BlockSpec((_NQ, blk, ch), lambda i: (0, i, 0)),
        ],
        out_shape=[out_sds, out_sds],
    )(xf, Wg, bg.reshape(1, C), Wf, bf.reshape(1, C))


def _sc_body(B, N, ch, rpt,
             e_hbm, f_hbm, ix_hbm, y0_hbm,
             ea_v, eb_v, fa_v, fb_v, d_v, z_v, idx_v, denom_sp, y0_sp,
             sem_ea, sem_eb, sem_fa, sem_fb, sem_d, sem_z,
             sem_o, sem_s1, sem_s2):
    c = lax.axis_index("c")
    s = lax.axis_index("s")
    base = s * rpt
    nu = 2 * B
    e_bufs, f_bufs = (ea_v, eb_v), (fa_v, fb_v)
    sem_e, sem_f = (sem_ea, sem_eb), (sem_fa, sem_fb)

    def row0(u):
        b, qq = u // 2, u % 2
        return ((2 * c + qq) * B + b) * N + base

    def zero_acc(acc):
        return [pltpu.async_copy(
                    z_v, acc.at[pl.ds(base + t * _ZROWS, _ZROWS)], sem_z)
                for t in range(rpt // _ZROWS)]

    pltpu.sync_copy(ix_hbm.at[pl.ds(base, rpt)], idx_v)

    zv16 = jnp.zeros((16,), jnp.float32)

    def _zero_row(r, _):
        for j in range(ch // 16):
            z_v[r, pl.ds(j * 16, 16)] = zv16
        return 0

    lax.fori_loop(0, _ZROWS, _zero_row, 0)

    he = pltpu.async_copy(e_hbm.at[pl.ds(row0(0), rpt), :], ea_v, sem_ea)
    hf = pltpu.async_copy(f_hbm.at[pl.ds(row0(0), rpt), :], fa_v, sem_fa)
    hzd = zero_acc(denom_sp)
    h_out = None

    for u in range(nu):
        e_v, f_v = e_bufs[u % 2], f_bufs[u % 2]

        def _vec_body(r, _):
            for j in range(ch // 16):
                sl = pl.ds(j * 16, 16)
                e = e_v[r, sl]
                d = d_v[r, sl]
                f = f_v[r, sl]
                e_v[r, sl] = f * (e / jnp.maximum(d, 1e-6))
            return 0

        if h_out is not None:
            h_out.wait()
        hzy = zero_acc(y0_sp)
        for h in hzd:
            h.wait()
        for h in hzy:
            h.wait()
        he.wait()

        plsc.subcore_barrier()

        pltpu.sync_copy(e_v, denom_sp.at[idx_v], add=True)

        plsc.subcore_barrier()

        hd = pltpu.async_copy(denom_sp.at[pl.ds(base, rpt)], d_v, sem_d)
        if u + 1 < nu:
            r1 = row0(u + 1)
            he = pltpu.async_copy(
                e_hbm.at[pl.ds(r1, rpt), :], e_bufs[(u + 1) % 2],
                sem_e[(u + 1) % 2])
            hf_n = pltpu.async_copy(
                f_hbm.at[pl.ds(r1, rpt), :], f_bufs[(u + 1) % 2],
                sem_f[(u + 1) % 2])
        hf.wait()
        hd.wait()

        hzd = []
        if u + 1 < nu:
            hf = hf_n
            hzd = zero_acc(denom_sp)

        lax.fori_loop(0, rpt, _vec_body, 0)

        pltpu.sync_copy(e_v, y0_sp.at[idx_v], add=True)

        plsc.subcore_barrier()
        pltpu.sync_copy(y0_sp.at[pl.ds(base, rpt)], d_v)
        h_out = pltpu.async_copy(
            d_v, y0_hbm.at[pl.ds(row0(u), rpt), :], sem_o)

    h_out.wait()


def _sc_stage(e2f, f2f, ix, B, N, ch):
    rpt = N // _NSUB
    mesh = plsc.VectorSubcoreMesh(
        core_axis_name="c", subcore_axis_name="s", num_cores=2,
        num_subcores=_NSUB)
    body = functools.partial(_sc_body, B, N, ch, rpt)
    sck = pl.kernel(
        body,
        mesh=mesh,
        out_type=jax.ShapeDtypeStruct((_NQ * B * N, ch), jnp.float32),
        scratch_types=[
            pltpu.VMEM((rpt, ch), jnp.float32),
            pltpu.VMEM((rpt, ch), jnp.float32),
            pltpu.VMEM((rpt, ch), jnp.float32),
            pltpu.VMEM((rpt, ch), jnp.float32),
            pltpu.VMEM((rpt, ch), jnp.float32),
            pltpu.VMEM((_ZROWS, ch), jnp.float32),
            pltpu.VMEM((rpt,), jnp.int32),
            pltpu.VMEM_SHARED((N, ch), jnp.float32),
            pltpu.VMEM_SHARED((N, ch), jnp.float32),
            pltpu.SemaphoreType.DMA,
            pltpu.SemaphoreType.DMA,
            pltpu.SemaphoreType.DMA,
            pltpu.SemaphoreType.DMA,
            pltpu.SemaphoreType.DMA,
            pltpu.SemaphoreType.DMA,
            pltpu.SemaphoreType.DMA,
            pltpu.SemaphoreType.DMA,
            pltpu.SemaphoreType.DMA,
        ],
    )
    return sck(e2f, f2f, ix)


def _stage2_body(y0_ref, wh_ref, bh_ref, o_ref):
    yb = jnp.concatenate([y0_ref[k] for k in range(_NQ)], axis=1)
    dn = (((1,), (1,)), ((), ()))
    o = lax.dot_general(yb, wh_ref[...], dn, preferred_element_type=jnp.float32)
    o_ref[...] = o + bh_ref[...]


def _stage2(y04, Wh, bh, blk):
    _, M, ch = y04.shape
    C = _NQ * ch
    grid = (M // blk,)
    return pl.pallas_call(
        _stage2_body,
        grid=grid,
        in_specs=[
            pl.BlockSpec((_NQ, blk, ch), lambda i: (0, i, 0)),
            pl.BlockSpec((C, C), lambda i: (0, 0)),
            pl.BlockSpec((1, C), lambda i: (0, 0)),
        ],
        out_specs=pl.BlockSpec((blk, C), lambda i: (i, 0)),
        out_shape=jax.ShapeDtypeStruct((M, C), jnp.float32),
    )(y04, Wh, bh.reshape(1, C))


def kernel(x, ix, Wf, bf, Wg, bg, Wh, bh):
    B, N, C = x.shape
    ch = C // _NQ
    blk = 512
    xf = x.reshape(B * N, C)
    e4, f4 = _stage1(xf, Wg, bg, Wf, bf, blk)
    e4f = e4.reshape(_NQ * B * N, ch)
    f4f = f4.reshape(_NQ * B * N, ch)
    y0f = _sc_stage(e4f, f4f, ix, B, N, ch)
    y04 = y0f.reshape(_NQ, B * N, ch)
    y = _stage2(y04, Wh, bh, blk)
    return y.reshape(B, N, C)
